# baseline (speedup 1.0000x reference)
import functools

import jax
import jax.numpy as jnp
from jax import lax
from jax.experimental import pallas as pl
from jax.experimental.pallas import tpu as pltpu
from jax.experimental.pallas import tpu_sc as plsc

_B = 16384
_D = 64
_NNEG = 20
_DOTS = _NNEG + 1
_NC = 2
_NS = 16
_NW = _NC * _NS
_BPW = _B // _NW
_G = 16
_NCHUNK = _BPW // _G
_GN = _G * _NNEG
_L = 16


def _sc_scores(target, context, negatives_flat, ttab2, ctab2):
    mesh = plsc.VectorSubcoreMesh(core_axis_name="c", subcore_axis_name="s")

    @functools.partial(
        pl.kernel,
        mesh=mesh,
        out_type=jax.ShapeDtypeStruct((_B * _DOTS * _L,), jnp.float32),
        scratch_types=[
            pltpu.VMEM((_G,), jnp.int32),
            pltpu.VMEM((_G,), jnp.int32),
            pltpu.VMEM((_GN,), jnp.int32),
            pltpu.VMEM((_G,), jnp.int32),
            pltpu.VMEM((_G,), jnp.int32),
            pltpu.VMEM((_GN,), jnp.int32),
            pltpu.VMEM((_G, 2 * _D), jnp.float32),
            pltpu.VMEM((_G, 2 * _D), jnp.float32),
            pltpu.VMEM((_GN, 2 * _D), jnp.float32),
            pltpu.VMEM((_G * _DOTS * _L,), jnp.float32),
            pltpu.SemaphoreType.DMA,
        ],
    )
    def k(tgt_hbm, ctx_hbm, neg_hbm, ttab, ctab, out_hbm,
          tidx, cidx, nidx, tidx2, cidx2, nidx2, trows, crows, nrows,
          obuf, sem):
        wid = lax.axis_index("s") * _NC + lax.axis_index("c")
        base = wid * _BPW

        def chunk_body(g, carry):
            rb = base + g * _G
            pltpu.sync_copy(tgt_hbm.at[pl.ds(rb, _G)], tidx)
            pltpu.sync_copy(ctx_hbm.at[pl.ds(rb, _G)], cidx)
            pltpu.sync_copy(neg_hbm.at[pl.ds(rb * _NNEG, _GN)], nidx)
            for jj in range(_G // _L):
                tidx2[pl.ds(jj * _L, _L)] = tidx[pl.ds(jj * _L, _L)] >> 1
                cidx2[pl.ds(jj * _L, _L)] = cidx[pl.ds(jj * _L, _L)] >> 1
            for jj in range(_GN // _L):
                nidx2[pl.ds(jj * _L, _L)] = nidx[pl.ds(jj * _L, _L)] >> 1
            pltpu.async_copy(ttab.at[tidx2], trows, sem).wait()
            pltpu.async_copy(ctab.at[cidx2], crows, sem).wait()
            for j in range(_GN // 128):
                pltpu.async_copy(
                    ctab.at[nidx2.at[pl.ds(j * 128, 128)]],
                    nrows.at[pl.ds(j * 128, 128)], sem).wait()

            tv = tidx[pl.ds(0, _L)]
            cv = cidx[pl.ds(0, _L)]
            for r in range(_G):
                to = (tv[r] & 1) * _D
                co = (cv[r] & 1) * _D
                t0 = trows[r, pl.ds(to, _L)]
                t1 = trows[r, pl.ds(to + 16, _L)]
                t2 = trows[r, pl.ds(to + 32, _L)]
                t3 = trows[r, pl.ds(to + 48, _L)]
                p = (t0 * crows[r, pl.ds(co, _L)]
                     + t1 * crows[r, pl.ds(co + 16, _L)]
                     + t2 * crows[r, pl.ds(co + 32, _L)]
                     + t3 * crows[r, pl.ds(co + 48, _L)])
                ob = r * (_DOTS * _L)
                obuf[pl.ds(ob, _L)] = p
                nv0 = nidx[pl.ds(r * _NNEG, _L)]
                nv1 = nidx[pl.ds(r * _NNEG + 4, _L)]
                for n in range(_NNEG):
                    m = r * _NNEG + n
                    no = ((nv0[n] if n < _L else nv1[n - 4]) & 1) * _D
                    q = (t0 * nrows[m, pl.ds(no, _L)]
                         + t1 * nrows[m, pl.ds(no + 16, _L)]
                         + t2 * nrows[m, pl.ds(no + 32, _L)]
                         + t3 * nrows[m, pl.ds(no + 48, _L)])
                    obuf[pl.ds(ob + (n + 1) * _L, _L)] = q

            pltpu.sync_copy(
                obuf, out_hbm.at[pl.ds(rb * _DOTS * _L, _G * _DOTS * _L)])
            return carry

        lax.fori_loop(0, _NCHUNK, chunk_body, 0)

    return k(target, context, negatives_flat, ttab2, ctab2)


_BM = 2048


def _tc_loss(partials):
    def body(p_ref, out_ref):
        i = pl.program_id(0)
        x = p_ref[...]
        k_iota = lax.broadcasted_iota(jnp.int32, (_DOTS * _L, 128), 0)
        n_iota = lax.broadcasted_iota(jnp.int32, (_DOTS * _L, 128), 1)
        m = ((k_iota // _L) == n_iota).astype(jnp.float32)
        s = jnp.dot(x, m, preferred_element_type=jnp.float32)
        col = lax.broadcasted_iota(jnp.int32, (_BM, 128), 1)
        signed = jnp.where(col == 0, s, -s)
        l = jnp.log(jax.nn.sigmoid(signed) + 1e-10)
        l = jnp.where(col < _DOTS, l, 0.0)
        part = jnp.sum(l)

        @pl.when(i == 0)
        def _():
            out_ref[0, 0] = 0.0

        out_ref[0, 0] += part

        @pl.when(i == pl.num_programs(0) - 1)
        def _():
            out_ref[0, 0] = out_ref[0, 0] * (-1.0 / _B)

    return pl.pallas_call(
        body,
        grid=(_B // _BM,),
        in_specs=[pl.BlockSpec((_BM, _DOTS * _L), lambda i: (i, 0))],
        out_specs=pl.BlockSpec(memory_space=pltpu.SMEM),
        out_shape=jax.ShapeDtypeStruct((1, 1), jnp.float32),
    )(partials)


def kernel(target, context, negatives, target_table, context_table):
    scores = _sc_scores(
        target.astype(jnp.int32),
        context.astype(jnp.int32),
        negatives.reshape(-1).astype(jnp.int32),
        target_table.reshape(-1, 2 * _D),
        context_table.reshape(-1, 2 * _D))
    loss = _tc_loss(scores.reshape(_B, _DOTS * _L))
    return loss[0, 0]


# speedup vs baseline: 1.1617x; 1.1617x over previous
import functools

import jax
import jax.numpy as jnp
from jax import lax
from jax.experimental import pallas as pl
from jax.experimental.pallas import tpu as pltpu
from jax.experimental.pallas import tpu_sc as plsc

_B = 16384
_D = 64
_NNEG = 20
_DOTS = _NNEG + 1
_NC = 2
_NS = 16
_NW = _NC * _NS
_BPW = _B // _NW
_G = 32
_NCHUNK = _BPW // _G
_LANES = 16


def _sc_scores(target, context, negatives_flat, target_table, context_table):
    mesh = plsc.VectorSubcoreMesh(core_axis_name="c", subcore_axis_name="s")

    @functools.partial(
        pl.kernel,
        mesh=mesh,
        compiler_params=pltpu.CompilerParams(use_tc_tiling_on_sc=False),
        out_type=jax.ShapeDtypeStruct((_B * _DOTS * _LANES,), jnp.float32),
        scratch_types=[
            pltpu.VMEM((_G,), jnp.int32),
            pltpu.VMEM((_G,), jnp.int32),
            pltpu.VMEM((_G * _NNEG,), jnp.int32),
            pltpu.VMEM((_G, _D), jnp.float32),
            pltpu.VMEM((_G, _D), jnp.float32),
            pltpu.VMEM((_G * _NNEG, _D), jnp.float32),
            pltpu.VMEM((_G * _DOTS * _LANES,), jnp.float32),
            pltpu.SemaphoreType.DMA,
        ],
    )
    def k(tgt_hbm, ctx_hbm, neg_hbm, ttab, ctab, out_hbm,
          tidx, cidx, nidx, trows, crows, nrows, obuf, sem):
        wid = lax.axis_index("s") * _NC + lax.axis_index("c")
        base = wid * _BPW

        def chunk_body(g, carry):
            rb = base + g * _G
            h1 = pltpu.async_copy(tgt_hbm.at[pl.ds(rb, _G)], tidx, sem)
            h2 = pltpu.async_copy(ctx_hbm.at[pl.ds(rb, _G)], cidx, sem)
            h3 = pltpu.async_copy(
                neg_hbm.at[pl.ds(rb * _NNEG, _G * _NNEG)], nidx, sem)
            h1.wait()
            h2.wait()
            h3.wait()
            g1 = pltpu.async_copy(ttab.at[tidx], trows, sem)
            g2 = pltpu.async_copy(ctab.at[cidx], crows, sem)
            gn = [
                pltpu.async_copy(
                    ctab.at[nidx.at[pl.ds(j * 128, 128)]],
                    nrows.at[pl.ds(j * 128, 128)], sem)
                for j in range(_G * _NNEG // 128)
            ]
            g1.wait()
            g2.wait()
            for h in gn:
                h.wait()

            def row_body(r, c2):
                t0 = trows[r, pl.ds(0, 16)]
                t1 = trows[r, pl.ds(16, 16)]
                t2 = trows[r, pl.ds(32, 16)]
                t3 = trows[r, pl.ds(48, 16)]
                p = (t0 * crows[r, pl.ds(0, 16)]
                     + t1 * crows[r, pl.ds(16, 16)]
                     + t2 * crows[r, pl.ds(32, 16)]
                     + t3 * crows[r, pl.ds(48, 16)])
                ob = r * (_DOTS * _LANES)
                obuf[pl.ds(ob, 16)] = p
                for n in range(_NNEG):
                    m = r * _NNEG + n
                    q = (t0 * nrows[m, pl.ds(0, 16)]
                         + t1 * nrows[m, pl.ds(16, 16)]
                         + t2 * nrows[m, pl.ds(32, 16)]
                         + t3 * nrows[m, pl.ds(48, 16)])
                    obuf[pl.ds(ob + (n + 1) * 16, 16)] = q
                return c2

            lax.fori_loop(0, _G, row_body, 0)
            pltpu.sync_copy(
                obuf,
                out_hbm.at[pl.ds(rb * _DOTS * _LANES, _G * _DOTS * _LANES)])
            return carry

        lax.fori_loop(0, _NCHUNK, chunk_body, 0)

    return k(target, context, negatives_flat, target_table, context_table)


_BM = 2048


def _tc_loss(partials):
    def body(p_ref, out_ref):
        i = pl.program_id(0)
        x = p_ref[...]
        k_iota = lax.broadcasted_iota(jnp.int32, (_DOTS * _LANES, 128), 0)
        n_iota = lax.broadcasted_iota(jnp.int32, (_DOTS * _LANES, 128), 1)
        m = ((k_iota // _LANES) == n_iota).astype(jnp.float32)
        s = jnp.dot(x, m, preferred_element_type=jnp.float32)
        col = lax.broadcasted_iota(jnp.int32, (_BM, 128), 1)
        signed = jnp.where(col == 0, s, -s)
        l = jnp.log(jax.nn.sigmoid(signed) + 1e-10)
        l = jnp.where(col < _DOTS, l, 0.0)
        part = jnp.sum(l)

        @pl.when(i == 0)
        def _():
            out_ref[0, 0] = 0.0

        out_ref[0, 0] += part

        @pl.when(i == pl.num_programs(0) - 1)
        def _():
            out_ref[0, 0] = out_ref[0, 0] * (-1.0 / _B)

    return pl.pallas_call(
        body,
        grid=(_B // _BM,),
        in_specs=[pl.BlockSpec((_BM, _DOTS * _LANES), lambda i: (i, 0))],
        out_specs=pl.BlockSpec(memory_space=pltpu.SMEM),
        out_shape=jax.ShapeDtypeStruct((1, 1), jnp.float32),
    )(partials)


def kernel(target, context, negatives, target_table, context_table):
    scores = _sc_scores(
        target.astype(jnp.int32),
        context.astype(jnp.int32),
        negatives.reshape(-1).astype(jnp.int32),
        target_table, context_table)
    loss = _tc_loss(scores.reshape(_B, _DOTS * _LANES))
    return loss[0, 0]
